# baseline (speedup 1.0000x reference)
import functools

import jax
import jax.numpy as jnp
from jax import lax
from jax.experimental import pallas as pl
from jax.experimental.pallas import tpu as pltpu
from jax.experimental.pallas import tpu_sc as plsc

E = 160000
N = 10000
H = 256
R = 6
HALF = 128
NC = 2
NS = 16
L = 16
PER_TILE = E // NS
B = 80
NCHUNK = PER_TILE // B
ROWS_PER_TILE = 624
ROWS_TAIL = N - NS * ROWS_PER_TILE
ZROWS = 16


def _sc_edge_scatter_body(x_hbm, rbft_hbm, i_hbm, wrbft_hbm, out_hbm,
                          x_v, h_v, rbf_v, idx_v, w_v, zbuf_v, acc_sh):
    c = lax.axis_index("c")
    s = lax.axis_index("s")
    coff = c * HALF

    pltpu.sync_copy(wrbft_hbm, w_v)

    def zbody(r, _):
        for v in range(HALF // L):
            zbuf_v[r, pl.ds(v * L, L)] = jnp.zeros((L,), jnp.float32)
        return 0
    lax.fori_loop(0, ZROWS, zbody, 0)

    def zcopy(j, _):
        pltpu.sync_copy(
            zbuf_v, acc_sh.at[pl.ds(s * ROWS_PER_TILE + j * ZROWS, ZROWS), :])
        return 0
    lax.fori_loop(0, ROWS_PER_TILE // ZROWS, zcopy, 0)

    @pl.when(s == NS - 1)
    def _zero_tail():
        pltpu.sync_copy(zbuf_v.at[pl.ds(0, ROWS_TAIL), :],
                        acc_sh.at[pl.ds(NS * ROWS_PER_TILE, ROWS_TAIL), :])
    plsc.subcore_barrier()

    w = [[w_v[r, pl.ds(coff + v * L, L)] for v in range(HALF // L)]
         for r in range(R)]

    def chunk_body(kk, _):
        e0 = s * PER_TILE + kk * B
        pltpu.sync_copy(x_hbm.at[pl.ds(e0, B), pl.ds(coff, HALF)], x_v)
        pltpu.sync_copy(rbft_hbm.at[pl.ds(e0, B), :], rbf_v)
        pltpu.sync_copy(i_hbm.at[pl.ds(e0, B)], idx_v)

        def ebody(e, _):
            rv = rbf_v[e, :]
            r0 = rv[0]
            r1 = rv[1]
            r2 = rv[2]
            r3 = rv[3]
            r4 = rv[4]
            r5 = rv[5]
            for v in range(HALF // L):
                xv = x_v[e, pl.ds(v * L, L)]
                acc = (r0 * w[0][v] + r1 * w[1][v] + r2 * w[2][v]
                       + r3 * w[3][v] + r4 * w[4][v] + r5 * w[5][v])
                h_v[e, pl.ds(v * L, L)] = acc * xv
            return 0
        lax.fori_loop(0, B, ebody, 0)

        pltpu.sync_copy(h_v, acc_sh.at[idx_v], add=True)
        return 0
    lax.fori_loop(0, NCHUNK, chunk_body, 0)

    plsc.subcore_barrier()
    pltpu.sync_copy(acc_sh.at[pl.ds(s * ROWS_PER_TILE, ROWS_PER_TILE), :],
                    out_hbm.at[pl.ds(s * ROWS_PER_TILE, ROWS_PER_TILE),
                               pl.ds(coff, HALF)])

    @pl.when(s == NS - 1)
    def _write_tail():
        pltpu.sync_copy(
            acc_sh.at[pl.ds(NS * ROWS_PER_TILE, ROWS_TAIL), :],
            out_hbm.at[pl.ds(NS * ROWS_PER_TILE, ROWS_TAIL), pl.ds(coff, HALF)])


def _sc_edge_scatter(x, rbf_t, idx, wrbf_t):
    return pl.kernel(
        _sc_edge_scatter_body,
        out_type=jax.ShapeDtypeStruct((N, H), jnp.float32),
        mesh=plsc.VectorSubcoreMesh(core_axis_name="c", subcore_axis_name="s"),
        scratch_types=[
            pltpu.VMEM((B, HALF), jnp.float32),
            pltpu.VMEM((B, HALF), jnp.float32),
            pltpu.VMEM((B, L), jnp.float32),
            pltpu.VMEM((B,), jnp.int32),
            pltpu.VMEM((R, H), jnp.float32),
            pltpu.VMEM((ZROWS, HALF), jnp.float32),
            pltpu.VMEM_SHARED((N, HALF), jnp.float32),
        ],
    )(x, rbf_t, idx, wrbf_t)


BLK = 1000


def _dot(a, b):
    return jnp.dot(a.astype(jnp.bfloat16), b.astype(jnp.bfloat16),
                   preferred_element_type=jnp.float32)


def _mlp_body(cnt_ref, h_ref, ww_ref, bw_ref, wup_ref, bup_ref,
              w0_ref, b0_ref, w1_ref, b1_ref, w2_ref, b2_ref, wout_ref,
              out_ref):
    h = h_ref[:, :] + cnt_ref[0]
    alpha = _dot(h, ww_ref[:, :]) + bw_ref[0]
    t = _dot(h, wup_ref[:, :]) + bup_ref[:, :]
    for w_ref, b_ref in ((w0_ref, b0_ref), (w1_ref, b1_ref), (w2_ref, b2_ref)):
        t = _dot(t, w_ref[:, :]) + b_ref[:, :]
        t = t * jax.nn.sigmoid(t)
    o = _dot(t, wout_ref[:, :])
    out_ref[:, :] = o * alpha


def _node_mlp(nodes, cnt, W_w, b_w, W_up, b_up, W_lin0, b_lin0,
              W_lin1, b_lin1, W_lin2, b_lin2, W_out):
    full = lambda shape: pl.BlockSpec(shape, lambda n: (0, 0))
    return pl.pallas_call(
        _mlp_body,
        grid=(N // BLK,),
        in_specs=[
            pl.BlockSpec(memory_space=pltpu.SMEM),
            pl.BlockSpec((BLK, H), lambda n: (n, 0)),
            full((H, 1)), pl.BlockSpec(memory_space=pltpu.SMEM),
            full((H, H)), full((1, H)),
            full((H, H)), full((1, H)),
            full((H, H)), full((1, H)),
            full((H, H)), full((1, H)),
            full((H, 1)),
        ],
        out_specs=pl.BlockSpec((BLK, 1), lambda n: (n, 0)),
        out_shape=jax.ShapeDtypeStruct((N, 1), jnp.float32),
    )(cnt, nodes, W_w.T, b_w, W_up.T, b_up.reshape(1, H),
      W_lin0.T, b_lin0.reshape(1, H), W_lin1.T, b_lin1.reshape(1, H),
      W_lin2.T, b_lin2.reshape(1, H), W_out.T)


def _round_bf16(v):
    u = jax.lax.bitcast_convert_type(v, jnp.uint32)
    u = (u + jnp.uint32(0x7FFF) + ((u >> 16) & jnp.uint32(1))) & jnp.uint32(0xFFFF0000)
    return jax.lax.bitcast_convert_type(u, jnp.float32)


def kernel(x, rbf, i, edge_index, edge_weight, batch, num_nodes,
           W_rbf, W_up, b_up, W_lin0, b_lin0, W_lin1, b_lin1,
           W_lin2, b_lin2, W_out, W_w, b_w):
    del edge_index, edge_weight
    idx = i.astype(jnp.int32)
    rbf_b = _round_bf16(rbf)
    wrbf_b = _round_bf16(W_rbf.T)
    rbf_pad = jnp.pad(rbf_b, ((0, 0), (0, L - R)))
    nodes = _sc_edge_scatter(x, rbf_pad, idx, wrbf_b)
    cnt = (jnp.asarray(num_nodes, jnp.float32) - jnp.float32(N)).reshape(1)
    out = _node_mlp(nodes, cnt, W_w, b_w, W_up, b_up, W_lin0, b_lin0,
                    W_lin1, b_lin1, W_lin2, b_lin2, W_out)
    return (out, batch)


# speedup vs baseline: 1.7482x; 1.7482x over previous
import jax
import jax.numpy as jnp
from jax import lax
from jax.experimental import pallas as pl
from jax.experimental.pallas import tpu as pltpu
from jax.experimental.pallas import tpu_sc as plsc

E = 160000
N = 10000
H = 256
R = 6
HALF = 128
NC = 2
NS = 16
L = 16
PER_TILE = E // NS
B = 40
NCHUNK = PER_TILE // B
ROWS_PER_TILE = 624
ROWS_TAIL = N - NS * ROWS_PER_TILE
ZROWS = 16


def _sc_edge_scatter_body(x_hbm, rbft_hbm, i_hbm, wrbft_hbm, out_hbm,
                          x_v0, x_v1, h_v0, h_v1, rbf_v0, rbf_v1,
                          idx_v0, idx_v1, idx_v2, idx_v3,
                          w_v, zbuf_v, acc_sh, sem0, sem1, ssc0, ssc1):
    c = lax.axis_index("c")
    s = lax.axis_index("s")
    coff = c * HALF

    pltpu.sync_copy(wrbft_hbm, w_v)

    def zbody(r, _):
        for v in range(HALF // L):
            zbuf_v[r, pl.ds(v * L, L)] = jnp.zeros((L,), jnp.float32)
        return 0
    lax.fori_loop(0, ZROWS, zbody, 0)

    def zcopy(j, _):
        pltpu.sync_copy(
            zbuf_v, acc_sh.at[pl.ds(s * ROWS_PER_TILE + j * ZROWS, ZROWS), :])
        return 0
    lax.fori_loop(0, ROWS_PER_TILE // ZROWS, zcopy, 0)

    @pl.when(s == NS - 1)
    def _zero_tail():
        pltpu.sync_copy(zbuf_v.at[pl.ds(0, ROWS_TAIL), :],
                        acc_sh.at[pl.ds(NS * ROWS_PER_TILE, ROWS_TAIL), :])
    plsc.subcore_barrier()

    w = [[w_v[r, pl.ds(coff + v * L, L)] for v in range(HALF // L)]
         for r in range(R)]

    def issue(kk, xv, rv, iv, sem):
        e0 = s * PER_TILE + kk * B
        pltpu.async_copy(x_hbm.at[pl.ds(e0, B), pl.ds(coff, HALF)], xv, sem)
        pltpu.async_copy(rbft_hbm.at[pl.ds(e0, B), :], rv, sem)
        pltpu.async_copy(i_hbm.at[pl.ds(e0, B)], iv, sem)

    def wait_in(xv, rv, iv, sem):
        pltpu.make_async_copy(x_hbm.at[pl.ds(0, B), pl.ds(0, HALF)], xv, sem).wait()
        pltpu.make_async_copy(rbft_hbm.at[pl.ds(0, B), :], rv, sem).wait()
        pltpu.make_async_copy(i_hbm.at[pl.ds(0, B)], iv, sem).wait()

    def work(kk, xv, hv, rv, iv, iv_next, sem, sem_sc):
        wait_in(xv, rv, iv, sem)

        @pl.when(kk >= 2)
        def _wait_sc():
            pltpu.make_async_copy(hv, acc_sh.at[iv], sem_sc).wait()

        def ebody(e, _):
            rvec = rv[e, :]
            r0 = rvec[0]
            r1 = rvec[1]
            r2 = rvec[2]
            r3 = rvec[3]
            r4 = rvec[4]
            r5 = rvec[5]
            for v in range(HALF // L):
                xvec = xv[e, pl.ds(v * L, L)]
                acc = (r0 * w[0][v] + r1 * w[1][v] + r2 * w[2][v]
                       + r3 * w[3][v] + r4 * w[4][v] + r5 * w[5][v])
                hv[e, pl.ds(v * L, L)] = acc * xvec
            return 0
        lax.fori_loop(0, B, ebody, 0)

        pltpu.async_copy(hv, acc_sh.at[iv], sem_sc, add=True)

        @pl.when(kk + 2 < NCHUNK)
        def _prefetch():
            issue(kk + 2, xv, rv, iv_next, sem)

    issue(0, x_v0, rbf_v0, idx_v0, sem0)
    issue(1, x_v1, rbf_v1, idx_v1, sem1)
    bufs = [
        (x_v0, h_v0, rbf_v0, idx_v0, idx_v2, sem0, ssc0),
        (x_v1, h_v1, rbf_v1, idx_v1, idx_v3, sem1, ssc1),
        (x_v0, h_v0, rbf_v0, idx_v2, idx_v0, sem0, ssc0),
        (x_v1, h_v1, rbf_v1, idx_v3, idx_v1, sem1, ssc1),
    ]

    def chunk_body(kk, _):
        for q in range(4):
            @pl.when(kk % 4 == q)
            def _q(q=q):
                work(kk, *bufs[q])
        return 0
    lax.fori_loop(0, NCHUNK, chunk_body, 0)

    pltpu.make_async_copy(h_v0, acc_sh.at[idx_v0], ssc0).wait()
    pltpu.make_async_copy(h_v1, acc_sh.at[idx_v1], ssc1).wait()

    plsc.subcore_barrier()
    pltpu.sync_copy(acc_sh.at[pl.ds(s * ROWS_PER_TILE, ROWS_PER_TILE), :],
                    out_hbm.at[pl.ds(s * ROWS_PER_TILE, ROWS_PER_TILE),
                               pl.ds(coff, HALF)])

    @pl.when(s == NS - 1)
    def _write_tail():
        pltpu.sync_copy(
            acc_sh.at[pl.ds(NS * ROWS_PER_TILE, ROWS_TAIL), :],
            out_hbm.at[pl.ds(NS * ROWS_PER_TILE, ROWS_TAIL), pl.ds(coff, HALF)])


def _sc_edge_scatter(x, rbf_t, idx, wrbf_t):
    return pl.kernel(
        _sc_edge_scatter_body,
        out_type=jax.ShapeDtypeStruct((N, H), jnp.float32),
        mesh=plsc.VectorSubcoreMesh(core_axis_name="c", subcore_axis_name="s"),
        scratch_types=[
            pltpu.VMEM((B, HALF), jnp.float32),
            pltpu.VMEM((B, HALF), jnp.float32),
            pltpu.VMEM((B, HALF), jnp.float32),
            pltpu.VMEM((B, HALF), jnp.float32),
            pltpu.VMEM((B, L), jnp.float32),
            pltpu.VMEM((B, L), jnp.float32),
            pltpu.VMEM((B,), jnp.int32),
            pltpu.VMEM((B,), jnp.int32),
            pltpu.VMEM((B,), jnp.int32),
            pltpu.VMEM((B,), jnp.int32),
            pltpu.VMEM((R, H), jnp.float32),
            pltpu.VMEM((ZROWS, HALF), jnp.float32),
            pltpu.VMEM_SHARED((N, HALF), jnp.float32),
            pltpu.SemaphoreType.DMA,
            pltpu.SemaphoreType.DMA,
            pltpu.SemaphoreType.DMA,
            pltpu.SemaphoreType.DMA,
        ],
    )(x, rbf_t, idx, wrbf_t)


BLK = 1000


def _dot(a, b):
    return jnp.dot(a.astype(jnp.bfloat16), b.astype(jnp.bfloat16),
                   preferred_element_type=jnp.float32)


def _mlp_body(cnt_ref, h_ref, ww_ref, bw_ref, wup_ref, bup_ref,
              w0_ref, b0_ref, w1_ref, b1_ref, w2_ref, b2_ref, wout_ref,
              out_ref):
    h = h_ref[:, :] + cnt_ref[0]
    alpha = _dot(h, ww_ref[:, :]) + bw_ref[0]
    t = _dot(h, wup_ref[:, :]) + bup_ref[:, :]
    for w_ref, b_ref in ((w0_ref, b0_ref), (w1_ref, b1_ref), (w2_ref, b2_ref)):
        t = _dot(t, w_ref[:, :]) + b_ref[:, :]
        t = t * jax.nn.sigmoid(t)
    o = _dot(t, wout_ref[:, :])
    out_ref[:, :] = o * alpha


def _node_mlp(nodes, cnt, W_w, b_w, W_up, b_up, W_lin0, b_lin0,
              W_lin1, b_lin1, W_lin2, b_lin2, W_out):
    full = lambda shape: pl.BlockSpec(shape, lambda n: (0, 0))
    return pl.pallas_call(
        _mlp_body,
        grid=(N // BLK,),
        in_specs=[
            pl.BlockSpec(memory_space=pltpu.SMEM),
            pl.BlockSpec((BLK, H), lambda n: (n, 0)),
            full((H, 1)), pl.BlockSpec(memory_space=pltpu.SMEM),
            full((H, H)), full((1, H)),
            full((H, H)), full((1, H)),
            full((H, H)), full((1, H)),
            full((H, H)), full((1, H)),
            full((H, 1)),
        ],
        out_specs=pl.BlockSpec((BLK, 1), lambda n: (n, 0)),
        out_shape=jax.ShapeDtypeStruct((N, 1), jnp.float32),
    )(cnt, nodes, W_w.T, b_w, W_up.T, b_up.reshape(1, H),
      W_lin0.T, b_lin0.reshape(1, H), W_lin1.T, b_lin1.reshape(1, H),
      W_lin2.T, b_lin2.reshape(1, H), W_out.T)


def _round_bf16(v):
    u = jax.lax.bitcast_convert_type(v, jnp.uint32)
    u = (u + jnp.uint32(0x7FFF) + ((u >> 16) & jnp.uint32(1))) & jnp.uint32(0xFFFF0000)
    return jax.lax.bitcast_convert_type(u, jnp.float32)


def kernel(x, rbf, i, edge_index, edge_weight, batch, num_nodes,
           W_rbf, W_up, b_up, W_lin0, b_lin0, W_lin1, b_lin1,
           W_lin2, b_lin2, W_out, W_w, b_w):
    del edge_index, edge_weight
    idx = i.astype(jnp.int32)
    rbf_b = _round_bf16(rbf)
    wrbf_b = _round_bf16(W_rbf.T)
    rbf_pad = jnp.pad(rbf_b, ((0, 0), (0, L - R)))
    nodes = _sc_edge_scatter(x, rbf_pad, idx, wrbf_b)
    cnt = (jnp.asarray(num_nodes, jnp.float32) - jnp.float32(N)).reshape(1)
    out = _node_mlp(nodes, cnt, W_w, b_w, W_up, b_up, W_lin0, b_lin0,
                    W_lin1, b_lin1, W_lin2, b_lin2, W_out)
    return (out, batch)
